# baseline (speedup 1.0000x reference)
import functools

import jax
import jax.numpy as jnp
from jax import lax
from jax.experimental import pallas as pl
from jax.experimental.pallas import tpu as pltpu
from jax.experimental.pallas import tpu_sc as plsc

N = 10000
NH = N // 2
D = 128
DH = 64
E = 320000

NC = 2
NS = 16

CHUNK = 128
NCH = 160
EPW = NCH * CHUNK
E_PAD = EPW * NS

NPACK = 5120
RCH = 40
NFL = NH // RCH
LANES = 16
HVECS = DH // LANES

NIDX = 4
UNROLL = 4


def _sc_aggregate_body(row_hbm, col_hbm, val_hbm, xp_hbm, out_hbm,
                       colv, rowv, valv, colh, rowh, rows_a, rows_b,
                       xpack, acc, sem_a, sem_b, sem_i):
    c = lax.axis_index("c")
    s = lax.axis_index("s")
    ebase = s * EPW
    bufs = (rows_a, rows_b)
    sems = (sem_a, sem_b)

    zvec = jnp.zeros((LANES,), jnp.float32)

    def _zero_buf(i, _):
        for l in range(D // LANES):
            rows_a[i, pl.ds(l * LANES, LANES)] = zvec
        return 0

    lax.fori_loop(0, RCH, _zero_buf, 0)

    for k in range((NFL + NS - 1) // NS):
        m = s + k * NS

        @pl.when(m < NFL)
        def _():
            sl = pl.ds(m * RCH, RCH)
            pltpu.sync_copy(rows_a.at[pl.ds(0, RCH)], acc.at[sl])
            pltpu.sync_copy(xp_hbm.at[pl.ds(c * NH + m * RCH, RCH)],
                            rows_b.at[pl.ds(0, RCH)])
            pltpu.sync_copy(rows_b.at[pl.ds(0, RCH)], xpack.at[sl])

    plsc.subcore_barrier()

    def _idx_start(j, p):
        sl = pl.ds(ebase + j * CHUNK, CHUNK)
        pltpu.make_async_copy(col_hbm.at[sl], colv[p], sem_i).start()
        pltpu.make_async_copy(row_hbm.at[sl], rowv[p], sem_i).start()
        pltpu.make_async_copy(val_hbm.at[sl], valv[p], sem_i).start()

    def _idx_wait(j, p):
        sl = pl.ds(ebase + j * CHUNK, CHUNK)
        pltpu.make_async_copy(col_hbm.at[sl], colv[p], sem_i).wait()
        pltpu.make_async_copy(row_hbm.at[sl], rowv[p], sem_i).wait()
        pltpu.make_async_copy(val_hbm.at[sl], valv[p], sem_i).wait()

    def _halve(p):
        for g in range(CHUNK // LANES):
            sl = pl.ds(g * LANES, LANES)
            colh[p][sl] = lax.shift_right_logical(colv[p][sl], 1)
            rowh[p][sl] = lax.shift_right_logical(rowv[p][sl], 1)

    def _gather_start(b, p):
        pltpu.make_async_copy(xpack.at[colh[p]], bufs[b], sems[b]).start()

    def _gather_wait(b, p):
        pltpu.make_async_copy(xpack.at[colh[p]], bufs[b], sems[b]).wait()

    _idx_start(0, 0)
    _idx_start(1, 1)
    _idx_start(2, 2)
    _idx_wait(0, 0)
    _halve(0)
    _gather_start(0, 0)

    def _edges(jj, _):
        for u in range(UNROLL):
            j = jj * UNROLL + u
            b = u % 2
            p = u % NIDX
            pn = (u + 1) % NIDX

            @pl.when(j + 1 < NCH)
            def _():
                _idx_wait(j + 1, pn)
                _halve(pn)
                _gather_start(1 - b, pn)

            _gather_wait(b, p)
            buf = bufs[b]

            def _scale(g, _):
                vv = valv[p][pl.ds(g * LANES, LANES)]
                cv = colv[p][pl.ds(g * LANES, LANES)]
                rv = rowv[p][pl.ds(g * LANES, LANES)]
                for e in range(LANES):
                    v = vv[e]
                    cp = (cv[e] & 1) * DH
                    rp = (rv[e] & 1) * DH
                    i = g * LANES + e
                    for l in range(HVECS):
                        x = buf[i, pl.ds(cp + l * LANES, LANES)] * v
                        buf[i, pl.ds(rp + l * LANES, LANES)] = x
                        buf[i, pl.ds((DH - rp) + l * LANES, LANES)] = zvec
                return 0

            lax.fori_loop(0, CHUNK // LANES, _scale, 0)

            pltpu.sync_copy(buf, acc.at[rowh[p]], add=True)

            @pl.when(j + 3 < NCH)
            def _():
                _idx_start(j + 3, (u + 3) % NIDX)
        return 0

    lax.fori_loop(0, NCH // UNROLL, _edges, 0)
    plsc.subcore_barrier()

    for k in range((NFL + NS - 1) // NS):
        m = s + k * NS

        @pl.when(m < NFL)
        def _():
            r0 = m * RCH
            pltpu.sync_copy(acc.at[pl.ds(r0, RCH)],
                            rows_a.at[pl.ds(0, RCH)])
            pltpu.sync_copy(rows_a.at[pl.ds(0, RCH)],
                            out_hbm.at[pl.ds(c * NH + r0, RCH)])


_sc_aggregate = functools.partial(
    pl.kernel,
    mesh=plsc.VectorSubcoreMesh(core_axis_name="c", subcore_axis_name="s"),
    out_type=jax.ShapeDtypeStruct((NC * NH, D), jnp.float32),
    scratch_types=[
        [pltpu.VMEM((CHUNK,), jnp.int32) for _ in range(NIDX)],
        [pltpu.VMEM((CHUNK,), jnp.int32) for _ in range(NIDX)],
        [pltpu.VMEM((CHUNK,), jnp.float32) for _ in range(NIDX)],
        [pltpu.VMEM((CHUNK,), jnp.int32) for _ in range(NIDX)],
        [pltpu.VMEM((CHUNK,), jnp.int32) for _ in range(NIDX)],
        pltpu.VMEM((CHUNK, D), jnp.float32),
        pltpu.VMEM((CHUNK, D), jnp.float32),
        pltpu.VMEM_SHARED((NH, D), jnp.float32),
        pltpu.VMEM_SHARED((NPACK, D), jnp.float32),
        pltpu.SemaphoreType.DMA,
        pltpu.SemaphoreType.DMA,
        pltpu.SemaphoreType.DMA,
    ],
)(_sc_aggregate_body)


BM = 1000


def _mm_body(p0_ref, p1_ref, a0_ref, a1_ref, b_ref, o_ref):
    o_ref[...] = (
        jnp.dot(p0_ref[...], a0_ref[...], preferred_element_type=jnp.float32)
        + jnp.dot(p1_ref[...], a1_ref[...], preferred_element_type=jnp.float32)
        + b_ref[...]
    )


def _tc_matmul(partial, a0, a1, biasp):
    return pl.pallas_call(
        _mm_body,
        grid=(NH // BM,),
        in_specs=[
            pl.BlockSpec((BM, D), lambda i: (i, 0)),
            pl.BlockSpec((BM, D), lambda i: (i + NH // BM, 0)),
            pl.BlockSpec((D, 2 * D), lambda i: (0, 0)),
            pl.BlockSpec((D, 2 * D), lambda i: (0, 0)),
            pl.BlockSpec((1, 2 * D), lambda i: (0, 0)),
        ],
        out_specs=pl.BlockSpec((BM, 2 * D), lambda i: (i, 0)),
        out_shape=jax.ShapeDtypeStruct((NH, 2 * D), jnp.float32),
    )(partial, partial, a0, a1, biasp)


def kernel(adj_indices, adj_values, input_feature, weight, bias):
    pad = E_PAD - E
    row = jnp.concatenate([adj_indices[0], jnp.zeros((pad,), jnp.int32)])
    col = jnp.concatenate([adj_indices[1], jnp.zeros((pad,), jnp.int32)])
    val = jnp.concatenate([adj_values, jnp.zeros((pad,), jnp.float32)])

    xp = jnp.concatenate(
        [input_feature[:, :DH].reshape(NH, D),
         input_feature[:, DH:].reshape(NH, D)], axis=0)

    partial = _sc_aggregate(row, col, val, xp)

    w0 = weight[:DH]
    w1 = weight[DH:]
    z = jnp.zeros((DH, D), jnp.float32)
    a0 = jnp.concatenate(
        [jnp.concatenate([w0, z], axis=1), jnp.concatenate([z, w0], axis=1)],
        axis=0)
    a1 = jnp.concatenate(
        [jnp.concatenate([w1, z], axis=1), jnp.concatenate([z, w1], axis=1)],
        axis=0)
    biasp = jnp.concatenate([bias, bias]).reshape(1, 2 * D)

    out_packed = _tc_matmul(partial, a0, a1, biasp)
    return out_packed.reshape(N, D)


# speedup vs baseline: 1.2256x; 1.2256x over previous
import functools

import jax
import jax.numpy as jnp
from jax import lax
from jax.experimental import pallas as pl
from jax.experimental.pallas import tpu as pltpu
from jax.experimental.pallas import tpu_sc as plsc

N = 10000
NH = N // 2
D = 128
DH = 64
E = 320000

NC = 2
NS = 16

CHUNK = 112
NCH = 180
EPW = NCH * CHUNK
E_PAD = EPW * NS

NPACK = NH
RCH = 40
NFL = NH // RCH
LANES = 16
HVECS = DH // LANES

NBUF = 3
NIDX = 4
UNROLL = 12


def _sc_aggregate_body(row_hbm, col_hbm, val_hbm, xp_hbm, out_hbm,
                       colv, rowv, valv, colh, rowh, bufs,
                       xpack, acc, gsems, ssems, sem_i):
    c = lax.axis_index("c")
    s = lax.axis_index("s")
    ebase = s * EPW
    rows_a = bufs[0]
    rows_b = bufs[1]

    zvec = jnp.zeros((LANES,), jnp.float32)

    def _zero_buf(i, _):
        for l in range(D // LANES):
            rows_a[i, pl.ds(l * LANES, LANES)] = zvec
        return 0

    lax.fori_loop(0, RCH, _zero_buf, 0)

    for k in range((NFL + NS - 1) // NS):
        m = s + k * NS

        @pl.when(m < NFL)
        def _():
            sl = pl.ds(m * RCH, RCH)
            pltpu.sync_copy(rows_a.at[pl.ds(0, RCH)], acc.at[sl])
            pltpu.sync_copy(xp_hbm.at[pl.ds(c * NH + m * RCH, RCH)],
                            rows_b.at[pl.ds(0, RCH)])
            pltpu.sync_copy(rows_b.at[pl.ds(0, RCH)], xpack.at[sl])

    plsc.subcore_barrier()

    def _idx_start(j, p):
        sl = pl.ds(ebase + j * CHUNK, CHUNK)
        pltpu.make_async_copy(col_hbm.at[sl], colv[p], sem_i).start()
        pltpu.make_async_copy(row_hbm.at[sl], rowv[p], sem_i).start()
        pltpu.make_async_copy(val_hbm.at[sl], valv[p], sem_i).start()

    def _idx_wait(j, p):
        sl = pl.ds(ebase + j * CHUNK, CHUNK)
        pltpu.make_async_copy(col_hbm.at[sl], colv[p], sem_i).wait()
        pltpu.make_async_copy(row_hbm.at[sl], rowv[p], sem_i).wait()
        pltpu.make_async_copy(val_hbm.at[sl], valv[p], sem_i).wait()

    def _halve(p):
        for g in range(CHUNK // LANES):
            sl = pl.ds(g * LANES, LANES)
            colh[p][sl] = lax.shift_right_logical(colv[p][sl], 1)
            rowh[p][sl] = lax.shift_right_logical(rowv[p][sl], 1)

    def _gather_start(b, p):
        pltpu.make_async_copy(xpack.at[colh[p]], bufs[b], gsems[b]).start()

    def _gather_wait(b, p):
        pltpu.make_async_copy(xpack.at[colh[p]], bufs[b], gsems[b]).wait()

    def _scat_start(b, p):
        pltpu.async_copy(bufs[b], acc.at[rowh[p]], ssems[b], add=True)

    def _scat_wait(b):
        pltpu.make_async_copy(bufs[b], acc.at[rowh[0]], ssems[b]).wait()

    _idx_start(0, 0)
    _idx_start(1, 1)
    _idx_start(2, 2)
    _idx_wait(0, 0)
    _halve(0)
    _gather_start(0, 0)

    def _edges(jj, _):
        for u in range(UNROLL):
            j = jj * UNROLL + u
            tb = u % NBUF
            tb1 = (u + 1) % NBUF
            p = u % NIDX
            pn = (u + 1) % NIDX

            @pl.when(j + 1 < NCH)
            def _():
                _idx_wait(j + 1, pn)
                _halve(pn)

            @pl.when((j >= 2) & (j + 1 < NCH))
            def _():
                _scat_wait(tb1)

            @pl.when(j + 1 < NCH)
            def _():
                _gather_start(tb1, pn)

            _gather_wait(tb, p)
            buf = bufs[tb]

            def _scale(g, _):
                vv = valv[p][pl.ds(g * LANES, LANES)]
                cv = colv[p][pl.ds(g * LANES, LANES)]
                rv = rowv[p][pl.ds(g * LANES, LANES)]
                for e in range(LANES):
                    v = vv[e]
                    cp = (cv[e] & 1) * DH
                    rp = (rv[e] & 1) * DH
                    i = g * LANES + e
                    for l in range(HVECS):
                        x = buf[i, pl.ds(cp + l * LANES, LANES)] * v
                        buf[i, pl.ds(rp + l * LANES, LANES)] = x
                        buf[i, pl.ds((DH - rp) + l * LANES, LANES)] = zvec
                return 0

            lax.fori_loop(0, CHUNK // LANES, _scale, 0)

            _scat_start(tb, p)

            @pl.when(j + 3 < NCH)
            def _():
                _idx_start(j + 3, (u + 3) % NIDX)
        return 0

    lax.fori_loop(0, NCH // UNROLL, _edges, 0)
    _scat_wait((NCH - 3) % NBUF)
    _scat_wait((NCH - 2) % NBUF)
    _scat_wait((NCH - 1) % NBUF)
    plsc.subcore_barrier()

    for k in range((NFL + NS - 1) // NS):
        m = s + k * NS

        @pl.when(m < NFL)
        def _():
            r0 = m * RCH
            pltpu.sync_copy(acc.at[pl.ds(r0, RCH)],
                            rows_a.at[pl.ds(0, RCH)])
            pltpu.sync_copy(rows_a.at[pl.ds(0, RCH)],
                            out_hbm.at[pl.ds(c * NH + r0, RCH)])


_sc_aggregate = functools.partial(
    pl.kernel,
    mesh=plsc.VectorSubcoreMesh(core_axis_name="c", subcore_axis_name="s"),
    out_type=jax.ShapeDtypeStruct((NC * NH, D), jnp.float32),
    scratch_types=[
        [pltpu.VMEM((CHUNK,), jnp.int32) for _ in range(NIDX)],
        [pltpu.VMEM((CHUNK,), jnp.int32) for _ in range(NIDX)],
        [pltpu.VMEM((CHUNK,), jnp.float32) for _ in range(NIDX)],
        [pltpu.VMEM((CHUNK,), jnp.int32) for _ in range(NIDX)],
        [pltpu.VMEM((CHUNK,), jnp.int32) for _ in range(NIDX)],
        [pltpu.VMEM((CHUNK, D), jnp.float32) for _ in range(NBUF)],
        pltpu.VMEM_SHARED((NH, D), jnp.float32),
        pltpu.VMEM_SHARED((NPACK, D), jnp.float32),
        [pltpu.SemaphoreType.DMA for _ in range(NBUF)],
        [pltpu.SemaphoreType.DMA for _ in range(NBUF)],
        pltpu.SemaphoreType.DMA,
    ],
)(_sc_aggregate_body)


BM = 1000


def _mm_body(p0_ref, p1_ref, a0_ref, a1_ref, b_ref, o_ref):
    o_ref[...] = (
        jnp.dot(p0_ref[...], a0_ref[...], preferred_element_type=jnp.float32)
        + jnp.dot(p1_ref[...], a1_ref[...], preferred_element_type=jnp.float32)
        + b_ref[...]
    )


def _tc_matmul(partial, a0, a1, biasp):
    return pl.pallas_call(
        _mm_body,
        grid=(NH // BM,),
        in_specs=[
            pl.BlockSpec((BM, D), lambda i: (i, 0)),
            pl.BlockSpec((BM, D), lambda i: (i + NH // BM, 0)),
            pl.BlockSpec((D, 2 * D), lambda i: (0, 0)),
            pl.BlockSpec((D, 2 * D), lambda i: (0, 0)),
            pl.BlockSpec((1, 2 * D), lambda i: (0, 0)),
        ],
        out_specs=pl.BlockSpec((BM, 2 * D), lambda i: (i, 0)),
        out_shape=jax.ShapeDtypeStruct((NH, 2 * D), jnp.float32),
    )(partial, partial, a0, a1, biasp)


def kernel(adj_indices, adj_values, input_feature, weight, bias):
    pad = E_PAD - E
    row = jnp.concatenate([adj_indices[0], jnp.zeros((pad,), jnp.int32)])
    col = jnp.concatenate([adj_indices[1], jnp.zeros((pad,), jnp.int32)])
    val = jnp.concatenate([adj_values, jnp.zeros((pad,), jnp.float32)])

    xp = jnp.concatenate(
        [input_feature[:, :DH].reshape(NH, D),
         input_feature[:, DH:].reshape(NH, D)], axis=0)

    partial = _sc_aggregate(row, col, val, xp)

    w0 = weight[:DH]
    w1 = weight[DH:]
    z = jnp.zeros((DH, D), jnp.float32)
    a0 = jnp.concatenate(
        [jnp.concatenate([w0, z], axis=1), jnp.concatenate([z, w0], axis=1)],
        axis=0)
    a1 = jnp.concatenate(
        [jnp.concatenate([w1, z], axis=1), jnp.concatenate([z, w1], axis=1)],
        axis=0)
    biasp = jnp.concatenate([bias, bias]).reshape(1, 2 * D)

    out_packed = _tc_matmul(partial, a0, a1, biasp)
    return out_packed.reshape(N, D)


# speedup vs baseline: 1.4051x; 1.1465x over previous
import functools

import jax
import jax.numpy as jnp
import numpy as np
from jax import lax
from jax.experimental import pallas as pl
from jax.experimental.pallas import tpu as pltpu
from jax.experimental.pallas import tpu_sc as plsc

N = 10000
D = 128
DW = 64
E = 320000

NC = 2
NS = 16
NW = NC * NS

CHUNK = 128
NCH = 80
EPW = NCH * CHUNK
E_PAD = EPW * NW

NP = 10240
FLUSH = 80
NFL = N // FLUSH
LANES = 16
WVECS = DW // LANES

NIDX = 4
UNROLL = 4

_PERM = np.empty((2 * DW,), np.int32)
for _p in range(DW):
    _PERM[2 * _p] = 32 * (_p // 16) + (_p % 16)
    _PERM[2 * _p + 1] = 32 * (_p // 16) + 16 + (_p % 16)


def _sc_aggregate_body(row_hbm, col_hbm, val_hbm, xw_hbm, out_hbm,
                       colv, rowv, valv, wbuf_a, wbuf_b, fbuf,
                       acc, sem_a, sem_b, sem_i):
    c = lax.axis_index("c")
    s = lax.axis_index("s")
    wid = s * NC + c
    ebase = wid * EPW
    bufs = (wbuf_a, wbuf_b)
    sems = (sem_a, sem_b)

    zvec = jnp.zeros((LANES,), jnp.float32)

    def _zero_buf(i, _):
        for l in range(D // LANES):
            fbuf[i, pl.ds(l * LANES, LANES)] = zvec
        return 0

    lax.fori_loop(0, FLUSH, _zero_buf, 0)

    for k in range((NFL + NS - 1) // NS):
        m = s + k * NS

        @pl.when(m < NFL)
        def _():
            pltpu.sync_copy(fbuf.at[pl.ds(0, FLUSH)],
                            acc.at[pl.ds(m * FLUSH, FLUSH)])

    plsc.subcore_barrier()

    def _idx_start(j, p):
        sl = pl.ds(ebase + j * CHUNK, CHUNK)
        pltpu.make_async_copy(col_hbm.at[sl], colv[p], sem_i).start()
        pltpu.make_async_copy(row_hbm.at[sl], rowv[p], sem_i).start()
        pltpu.make_async_copy(val_hbm.at[sl], valv[p], sem_i).start()

    def _idx_wait(j, p):
        sl = pl.ds(ebase + j * CHUNK, CHUNK)
        pltpu.make_async_copy(col_hbm.at[sl], colv[p], sem_i).wait()
        pltpu.make_async_copy(row_hbm.at[sl], rowv[p], sem_i).wait()
        pltpu.make_async_copy(val_hbm.at[sl], valv[p], sem_i).wait()

    def _gather_start(b, p):
        pltpu.make_async_copy(xw_hbm.at[colv[p]], bufs[b], sems[b]).start()

    def _gather_wait(b, p):
        pltpu.make_async_copy(xw_hbm.at[colv[p]], bufs[b], sems[b]).wait()

    _idx_start(0, 0)
    _idx_start(1, 1)
    _idx_start(2, 2)
    _idx_wait(0, 0)
    _gather_start(0, 0)

    mask_hi = jnp.int32(-65536)

    def _edges(jj, _):
        for u in range(UNROLL):
            j = jj * UNROLL + u
            b = u % 2
            p = u % NIDX
            pn = (u + 1) % NIDX

            @pl.when(j + 1 < NCH)
            def _():
                _idx_wait(j + 1, pn)
                _gather_start(1 - b, pn)

            _gather_wait(b, p)
            wbuf = bufs[b]

            def _scale(g, _):
                vv = valv[p][pl.ds(g * LANES, LANES)]
                for e in range(LANES):
                    v = vv[e]
                    i = g * LANES + e
                    for l in range(WVECS):
                        w = wbuf[i, pl.ds(l * LANES, LANES)]
                        even = plsc.bitcast(
                            lax.shift_left(w, 16), jnp.float32)
                        odd = plsc.bitcast(w & mask_hi, jnp.float32)
                        fbuf[i, pl.ds(2 * l * LANES, LANES)] = even * v
                        fbuf[i, pl.ds((2 * l + 1) * LANES, LANES)] = odd * v
                return 0

            lax.fori_loop(0, CHUNK // LANES, _scale, 0)

            pltpu.sync_copy(fbuf, acc.at[rowv[p]], add=True)

            @pl.when(j + 3 < NCH)
            def _():
                _idx_start(j + 3, (u + 3) % NIDX)
        return 0

    lax.fori_loop(0, NCH // UNROLL, _edges, 0)
    plsc.subcore_barrier()

    for k in range((NFL + NS - 1) // NS):
        m = s + k * NS

        @pl.when(m < NFL)
        def _():
            r0 = m * FLUSH
            pltpu.sync_copy(acc.at[pl.ds(r0, FLUSH)],
                            fbuf.at[pl.ds(0, FLUSH)])
            pltpu.sync_copy(fbuf.at[pl.ds(0, FLUSH)],
                            out_hbm.at[pl.ds(c * NP + r0, FLUSH)])


_sc_aggregate = functools.partial(
    pl.kernel,
    mesh=plsc.VectorSubcoreMesh(core_axis_name="c", subcore_axis_name="s"),
    out_type=jax.ShapeDtypeStruct((NC * NP, D), jnp.float32),
    compiler_params=pltpu.CompilerParams(needs_layout_passes=False,
                                         use_tc_tiling_on_sc=False),
    scratch_types=[
        [pltpu.VMEM((CHUNK,), jnp.int32) for _ in range(NIDX)],
        [pltpu.VMEM((CHUNK,), jnp.int32) for _ in range(NIDX)],
        [pltpu.VMEM((CHUNK,), jnp.float32) for _ in range(NIDX)],
        pltpu.VMEM((CHUNK, DW), jnp.int32),
        pltpu.VMEM((CHUNK, DW), jnp.int32),
        pltpu.VMEM((CHUNK, D), jnp.float32),
        pltpu.VMEM_SHARED((NP, D), jnp.float32),
        pltpu.SemaphoreType.DMA,
        pltpu.SemaphoreType.DMA,
        pltpu.SemaphoreType.DMA,
    ],
)(_sc_aggregate_body)


BM = 80


def _mm_body(p0_ref, p1_ref, w_ref, b_ref, o_ref):
    x = p0_ref[...] + p1_ref[...]
    o_ref[...] = (
        jnp.dot(x, w_ref[...], preferred_element_type=jnp.float32) + b_ref[...]
    )


def _tc_matmul(partial, weight, bias2d):
    return pl.pallas_call(
        _mm_body,
        grid=(N // BM,),
        in_specs=[
            pl.BlockSpec((BM, D), lambda i: (i, 0)),
            pl.BlockSpec((BM, D), lambda i: (i + NP // BM, 0)),
            pl.BlockSpec((D, D), lambda i: (0, 0)),
            pl.BlockSpec((1, D), lambda i: (0, 0)),
        ],
        out_specs=pl.BlockSpec((BM, D), lambda i: (i, 0)),
        out_shape=jax.ShapeDtypeStruct((N, D), jnp.float32),
    )(partial, partial, weight, bias2d)


def kernel(adj_indices, adj_values, input_feature, weight, bias):
    pad = E_PAD - E
    row = jnp.concatenate([adj_indices[0], jnp.zeros((pad,), jnp.int32)])
    col = jnp.concatenate([adj_indices[1], jnp.zeros((pad,), jnp.int32)])
    val = jnp.concatenate([adj_values, jnp.zeros((pad,), jnp.float32)])

    xb = input_feature[:, jnp.asarray(_PERM)].astype(jnp.bfloat16)
    xw = lax.bitcast_convert_type(xb.reshape(N, DW, 2), jnp.int32)

    partial = _sc_aggregate(row, col, val, xw)
    return _tc_matmul(partial, weight, bias.reshape(1, D))
